# Initial kernel scaffold; baseline (speedup 1.0000x reference)
import functools

import jax
import jax.numpy as jnp
from jax import lax
from jax.experimental import pallas as pl
from jax.experimental.pallas import tpu as pltpu
from jax.experimental.pallas import tpu_sc as plsc

F = 26
B = 4096
L = 20
V = 100000
D = 16

NC = 2
NS = 16
NW = NC * NS

BAGS = B * F
BAGS_PER_W = BAGS // NW
R = 64
CHUNKS = BAGS_PER_W // R
IDX_PER_CHUNK = R * L
GROWS = 128
NG = IDX_PER_CHUNK // GROWS

_mesh = plsc.VectorSubcoreMesh(
    core_axis_name="c", subcore_axis_name="s", num_cores=NC, num_subcores=NS
)


@functools.partial(
    pl.kernel,
    out_type=jax.ShapeDtypeStruct((BAGS, D), jnp.float32),
    mesh=_mesh,
    scratch_types=[
        pltpu.VMEM((NG, GROWS), jnp.int32),
        pltpu.VMEM((IDX_PER_CHUNK, D), jnp.float32),
        pltpu.VMEM((R, D), jnp.float32),
        pltpu.SemaphoreType.DMA,
    ],
)
def _pooled_gather(tab_hbm, idx_hbm, out_hbm, idx_v, g_v, o_v, sem):
    wid = lax.axis_index("s") * NC + lax.axis_index("c")

    @pl.loop(0, CHUNKS)
    def _chunk(c):
        row0 = (wid * CHUNKS + c) * NG
        pltpu.sync_copy(idx_hbm.at[pl.ds(row0, NG)], idx_v)
        copies = [
            pltpu.async_copy(
                tab_hbm.at[idx_v.at[j]], g_v.at[pl.ds(j * GROWS, GROWS)], sem
            )
            for j in range(NG)
        ]
        for cp in copies:
            cp.wait()

        @pl.loop(0, R)
        def _bag(b):
            base = b * L
            acc = g_v[base, :]
            for l in range(1, L):
                acc = acc + g_v[base + l, :]
            o_v[b, :] = acc

        pltpu.sync_copy(o_v, out_hbm.at[pl.ds(wid * BAGS_PER_W + c * R, R)])


def kernel(indices, tables):
    flat_tables = tables.reshape(F * V, D)
    offs = (jnp.arange(F, dtype=jnp.int32) * V)[None, :, None]
    flat_idx = (jnp.transpose(indices, (1, 0, 2)) + offs).reshape(
        BAGS * L // GROWS, GROWS
    )
    pooled = _pooled_gather(flat_tables, flat_idx)
    return pooled.reshape(B, F * D)


# speedup vs baseline: 4.1871x; 4.1871x over previous
import functools

import jax
import jax.numpy as jnp
from jax import lax
from jax.experimental import pallas as pl
from jax.experimental.pallas import tpu as pltpu
from jax.experimental.pallas import tpu_sc as plsc

F = 26
B = 4096
L = 20
V = 100000
D = 16

NC = 2
NS = 16
NW = NC * NS

BAGS = B * F
BAGS_PER_W = BAGS // NW
R = 256
CHUNKS = BAGS_PER_W // R
IDX_PER_CHUNK = R * L
GROWS = 128
NG = IDX_PER_CHUNK // GROWS
WIN = 5
NWIN = NG // WIN
BAGS_PER_WIN = WIN * GROWS // L

_mesh = plsc.VectorSubcoreMesh(
    core_axis_name="c", subcore_axis_name="s", num_cores=NC, num_subcores=NS
)


@functools.partial(
    pl.kernel,
    out_type=jax.ShapeDtypeStruct((BAGS, D), jnp.float32),
    mesh=_mesh,
    scratch_types=[
        pltpu.VMEM((NG, GROWS), jnp.int32),
        pltpu.VMEM((IDX_PER_CHUNK, D), jnp.float32),
        pltpu.VMEM((R, D), jnp.float32),
        pltpu.SemaphoreType.DMA((NWIN,)),
    ],
    compiler_params=pltpu.CompilerParams(use_tc_tiling_on_sc=False),
)
def _pooled_gather(tab_hbm, idx_hbm, out_hbm, idx_v, g_v, o_v, sems):
    wid = lax.axis_index("s") * NC + lax.axis_index("c")

    @pl.loop(0, CHUNKS)
    def _chunk(c):
        row0 = pl.multiple_of((wid * CHUNKS + c) * NG, 8)
        pltpu.sync_copy(idx_hbm.at[pl.ds(row0, NG)], idx_v)
        copies = [
            pltpu.async_copy(
                tab_hbm.at[idx_v.at[j]],
                g_v.at[pl.ds(j * GROWS, GROWS)],
                sems.at[j // WIN],
            )
            for j in range(NG)
        ]
        for w in range(NWIN):
            for cp in copies[w * WIN : (w + 1) * WIN]:
                cp.wait()

            @pl.loop(0, BAGS_PER_WIN)
            def _bag(b):
                base = (w * BAGS_PER_WIN + b) * L
                acc = g_v[base, :]
                for l in range(1, L):
                    acc = acc + g_v[base + l, :]
                o_v[w * BAGS_PER_WIN + b, :] = acc

        out0 = pl.multiple_of(wid * BAGS_PER_W + c * R, 8)
        pltpu.sync_copy(o_v, out_hbm.at[pl.ds(out0, R)])


def kernel(indices, tables):
    flat_tables = tables.reshape(F * V, D)
    offs = (jnp.arange(F, dtype=jnp.int32) * V)[None, :, None]
    flat_idx = (jnp.transpose(indices, (1, 0, 2)) + offs).reshape(
        BAGS * L // GROWS, GROWS
    )
    pooled = _pooled_gather(flat_tables, flat_idx)
    return pooled.reshape(B, F * D)
